# baseline (speedup 1.0000x reference)
import functools

import jax
import jax.numpy as jnp
from jax import lax
from jax.experimental import pallas as pl
from jax.experimental.pallas import tpu as pltpu
from jax.experimental.pallas import tpu_sc as plsc

B = 1024
N = 200
NP = 208
NH = NP // 2
FEAT = 64
PAD_ID = -1
VOCAB = 100000
SENT_S = VOCAB
SENT_D = VOCAB + 8
HIST = VOCAB + 16
CHUNKS = NP // 16
BB = 64
G = 16

_info = plsc.get_sparse_core_info()
_NC, _NS = _info.num_cores, _info.num_subcores
NW = _NC * _NS
ROWS_PER_W = B // NW


def _sc_count_kernel(nrows, src_hbm, dst_hbm, css_hbm, csd_hbm, cdd_hbm,
                     cds_hbm, hist, sbuf, dbuf, o_ss, o_sd, o_dd, o_ds):
    rows_per_w = nrows // NW
    wid = lax.axis_index("s") * _NC + lax.axis_index("c")
    base = wid * rows_per_w

    def zero_body(i, carry):
        hist[pl.ds(i * 16, 16)] = jnp.zeros((16,), jnp.int32)
        return carry

    lax.fori_loop(0, HIST // 16, zero_body, 0)

    ones16 = jnp.ones((16,), jnp.int32)
    zeros16 = jnp.zeros((16,), jnp.int32)

    def group_body(g, carry):
        row0 = base + g * G
        pltpu.sync_copy(src_hbm.at[pl.ds(row0, G)], sbuf)
        pltpu.sync_copy(dst_hbm.at[pl.ds(row0, G)], dbuf)
        for j in range(G):
            for k in range(CHUNKS):
                plsc.addupdate_scatter(
                    hist, [sbuf[j, pl.ds(k * 16, 16)]], ones16)
            for k in range(CHUNKS):
                o_ss[j, pl.ds(k * 16, 16)] = plsc.load_gather(
                    hist, [sbuf[j, pl.ds(k * 16, 16)]])
                o_ds[j, pl.ds(k * 16, 16)] = plsc.load_gather(
                    hist, [dbuf[j, pl.ds(k * 16, 16)]])
            for k in range(CHUNKS):
                plsc.store_scatter(hist, [sbuf[j, pl.ds(k * 16, 16)]], zeros16)
            for k in range(CHUNKS):
                plsc.addupdate_scatter(
                    hist, [dbuf[j, pl.ds(k * 16, 16)]], ones16)
            for k in range(CHUNKS):
                o_dd[j, pl.ds(k * 16, 16)] = plsc.load_gather(
                    hist, [dbuf[j, pl.ds(k * 16, 16)]])
                o_sd[j, pl.ds(k * 16, 16)] = plsc.load_gather(
                    hist, [sbuf[j, pl.ds(k * 16, 16)]])
            for k in range(CHUNKS):
                plsc.store_scatter(hist, [dbuf[j, pl.ds(k * 16, 16)]], zeros16)
        pltpu.sync_copy(o_ss, css_hbm.at[pl.ds(row0, G)])
        pltpu.sync_copy(o_sd, csd_hbm.at[pl.ds(row0, G)])
        pltpu.sync_copy(o_dd, cdd_hbm.at[pl.ds(row0, G)])
        pltpu.sync_copy(o_ds, cds_hbm.at[pl.ds(row0, G)])
        return carry

    lax.fori_loop(0, rows_per_w // G, group_body, 0)


def _sc_counts(src_p, dst_p):
    nrows = src_p.shape[0]
    mesh = plsc.VectorSubcoreMesh(core_axis_name="c", subcore_axis_name="s")
    c_t = jax.ShapeDtypeStruct((nrows, NP), jnp.int32)
    buf = pltpu.VMEM((G, NP), jnp.int32)
    f = pl.kernel(
        functools.partial(_sc_count_kernel, nrows),
        mesh=mesh,
        compiler_params=pltpu.CompilerParams(needs_layout_passes=False),
        out_type=[c_t, c_t, c_t, c_t],
        scratch_types=[
            pltpu.VMEM((HIST,), jnp.int32),
            buf, buf, buf, buf, buf, buf,
        ],
    )
    return f(src_p, dst_p)


def _encode_tc_kernel(css_ref, csd_ref, cdd_ref, cds_ref, srcp_ref, dstp_ref,
                      w1_ref, b1_ref, w2d_ref, b2_ref,
                      out_src_ref, out_dst_ref):
    s_pad = srcp_ref[...] == PAD_ID
    d_pad = dstp_ref[...] == PAD_ID
    zero = jnp.zeros((BB, NP), jnp.float32)
    c_ss = jnp.where(s_pad, zero, css_ref[...].astype(jnp.float32))
    c_sd = jnp.where(s_pad, zero, csd_ref[...].astype(jnp.float32))
    c_dd = jnp.where(d_pad, zero, cdd_ref[...].astype(jnp.float32))
    c_ds = jnp.where(d_pad, zero, cds_ref[...].astype(jnp.float32))

    w1 = w1_ref[0, :].astype(jnp.bfloat16)
    b1 = b1_ref[0, :].astype(jnp.bfloat16)
    w2d = w2d_ref[...]
    b2 = b2_ref[0, :]
    b2c = jnp.concatenate([b2, b2])

    def encode(c_self, c_cross, out_ref):
        a1 = c_self.astype(jnp.bfloat16)[:, :, None] * w1 + b1
        a2 = c_cross.astype(jnp.bfloat16)[:, :, None] * w1 + b1
        h = jax.nn.relu(jnp.concatenate(
            [a1[:, :NH], a2[:, :NH], a1[:, NH:], a2[:, NH:]],
            axis=2))
        y = jnp.dot(h.reshape(BB * NH, 4 * FEAT), w2d,
                    preferred_element_type=jnp.float32)
        y = y + 2.0 * b2c[None, :]
        out_ref[...] = y.reshape(BB, NH, 2 * FEAT)[:, :N // 2, :]

    encode(c_ss, c_sd, out_src_ref)
    encode(c_dd, c_ds, out_dst_ref)


def _tc_encode(css, csd, cdd, cds, src_p, dst_p, w1, b1, w2d, b2):
    nrows = css.shape[0]
    grid = nrows // BB
    cspec = pl.BlockSpec((BB, NP), lambda i: (i, 0))
    wspec = pl.BlockSpec((1, FEAT), lambda i: (0, 0))
    out_shape = [
        jax.ShapeDtypeStruct((nrows, N // 2, 2 * FEAT), jnp.float32),
        jax.ShapeDtypeStruct((nrows, N // 2, 2 * FEAT), jnp.float32),
    ]
    f = pl.pallas_call(
        _encode_tc_kernel,
        grid=(grid,),
        in_specs=[cspec, cspec, cspec, cspec, cspec, cspec,
                  wspec, wspec,
                  pl.BlockSpec((4 * FEAT, 2 * FEAT), lambda i: (0, 0)),
                  wspec],
        out_specs=[
            pl.BlockSpec((BB, N // 2, 2 * FEAT), lambda i: (i, 0, 0)),
            pl.BlockSpec((BB, N // 2, 2 * FEAT), lambda i: (i, 0, 0)),
        ],
        out_shape=out_shape,
    )
    o_s, o_d = f(css, csd, cdd, cds, src_p, dst_p, w1, b1, w2d, b2)
    return o_s.reshape(nrows, N, FEAT), o_d.reshape(nrows, N, FEAT)


@jax.jit
def _run(src, dst, w1, b1, w2, b2):
    pad_s = jnp.full((B, NP - N), SENT_S, jnp.int32)
    pad_d = jnp.full((B, NP - N), SENT_D, jnp.int32)
    perm = jnp.concatenate([jnp.arange(0, NP, 2), jnp.arange(1, NP, 2)])
    src_p = jnp.concatenate([src, pad_s], axis=1)[:, perm]
    dst_p = jnp.concatenate([dst, pad_d], axis=1)[:, perm]
    w2s = jnp.concatenate([w2, w2], axis=0).astype(jnp.bfloat16)
    zbk = jnp.zeros((2 * FEAT, FEAT), jnp.bfloat16)
    w2d = jnp.concatenate(
        [jnp.concatenate([w2s, zbk], axis=1),
         jnp.concatenate([zbk, w2s], axis=1)], axis=0)
    hb = B // 2
    ca = _sc_counts(src_p[:hb], dst_p[:hb])
    cb = _sc_counts(src_p[hb:], dst_p[hb:])
    oa = _tc_encode(*ca, src_p[:hb], dst_p[:hb], w1, b1, w2d, b2)
    ob = _tc_encode(*cb, src_p[hb:], dst_p[hb:], w1, b1, w2d, b2)
    return (jnp.concatenate([oa[0], ob[0]], axis=0),
            jnp.concatenate([oa[1], ob[1]], axis=0))


def kernel(src_neighbour_nodes_ids, dst_neighbour_nodes_ids, W1, b1, W2, b2):
    w1 = W1.reshape(1, FEAT)
    b1r = b1.reshape(1, FEAT)
    b2r = b2.reshape(1, FEAT)
    out_s, out_d = _run(src_neighbour_nodes_ids, dst_neighbour_nodes_ids,
                        w1, b1r, W2, b2r)
    return (out_s, out_d)


# speedup vs baseline: 1.2236x; 1.2236x over previous
import functools

import jax
import jax.numpy as jnp
from jax import lax
from jax.experimental import pallas as pl
from jax.experimental.pallas import tpu as pltpu
from jax.experimental.pallas import tpu_sc as plsc

B = 1024
N = 200
NP = 208
NH = NP // 2
FEAT = 64
PAD_ID = -1
VOCAB = 100000
SENT_S = VOCAB
SENT_D = VOCAB + 8
HIST = VOCAB + 16
CHUNKS = NP // 16
BB = 64
G = 16

_info = plsc.get_sparse_core_info()
_NC, _NS = _info.num_cores, _info.num_subcores
NW = _NC * _NS
ROWS_PER_W = B // NW


def _sc_count_kernel(src_hbm, dst_hbm, css_hbm, csd_hbm, cdd_hbm, cds_hbm,
                     hist, sbuf, dbuf, o_ss, o_sd, o_dd, o_ds):
    wid = lax.axis_index("s") * _NC + lax.axis_index("c")
    base = wid * ROWS_PER_W

    def zero_body(i, carry):
        hist[pl.ds(i * 16, 16)] = jnp.zeros((16,), jnp.int32)
        return carry

    lax.fori_loop(0, HIST // 16, zero_body, 0)

    ones16 = jnp.ones((16,), jnp.int32)
    zeros16 = jnp.zeros((16,), jnp.int32)

    def group_body(g, carry):
        row0 = base + g * G
        pltpu.sync_copy(src_hbm.at[pl.ds(row0, G)], sbuf)
        pltpu.sync_copy(dst_hbm.at[pl.ds(row0, G)], dbuf)
        for j in range(G):
            for k in range(CHUNKS):
                plsc.addupdate_scatter(
                    hist, [sbuf[j, pl.ds(k * 16, 16)]], ones16)
            for k in range(CHUNKS):
                o_ss[j, pl.ds(k * 16, 16)] = plsc.load_gather(
                    hist, [sbuf[j, pl.ds(k * 16, 16)]])
                o_ds[j, pl.ds(k * 16, 16)] = plsc.load_gather(
                    hist, [dbuf[j, pl.ds(k * 16, 16)]])
            for k in range(CHUNKS):
                plsc.store_scatter(hist, [sbuf[j, pl.ds(k * 16, 16)]], zeros16)
            for k in range(CHUNKS):
                plsc.addupdate_scatter(
                    hist, [dbuf[j, pl.ds(k * 16, 16)]], ones16)
            for k in range(CHUNKS):
                o_dd[j, pl.ds(k * 16, 16)] = plsc.load_gather(
                    hist, [dbuf[j, pl.ds(k * 16, 16)]])
                o_sd[j, pl.ds(k * 16, 16)] = plsc.load_gather(
                    hist, [sbuf[j, pl.ds(k * 16, 16)]])
            for k in range(CHUNKS):
                plsc.store_scatter(hist, [dbuf[j, pl.ds(k * 16, 16)]], zeros16)
        pltpu.sync_copy(o_ss, css_hbm.at[pl.ds(row0, G)])
        pltpu.sync_copy(o_sd, csd_hbm.at[pl.ds(row0, G)])
        pltpu.sync_copy(o_dd, cdd_hbm.at[pl.ds(row0, G)])
        pltpu.sync_copy(o_ds, cds_hbm.at[pl.ds(row0, G)])
        return carry

    lax.fori_loop(0, ROWS_PER_W // G, group_body, 0)


def _sc_counts(src_p, dst_p):
    mesh = plsc.VectorSubcoreMesh(core_axis_name="c", subcore_axis_name="s")
    c_t = jax.ShapeDtypeStruct((B, NP), jnp.int32)
    buf = pltpu.VMEM((G, NP), jnp.int32)
    f = pl.kernel(
        _sc_count_kernel,
        mesh=mesh,
        compiler_params=pltpu.CompilerParams(needs_layout_passes=False),
        out_type=[c_t, c_t, c_t, c_t],
        scratch_types=[
            pltpu.VMEM((HIST,), jnp.int32),
            buf, buf, buf, buf, buf, buf,
        ],
    )
    return f(src_p, dst_p)


def _encode_tc_kernel(css_ref, csd_ref, cdd_ref, cds_ref, srcp_ref, dstp_ref,
                      w1_ref, b1_ref, w2d_ref, b2_ref,
                      out_src_ref, out_dst_ref):
    s_pad = srcp_ref[...] == PAD_ID
    d_pad = dstp_ref[...] == PAD_ID
    zero = jnp.zeros((BB, NP), jnp.float32)
    c_ss = jnp.where(s_pad, zero, css_ref[...].astype(jnp.float32))
    c_sd = jnp.where(s_pad, zero, csd_ref[...].astype(jnp.float32))
    c_dd = jnp.where(d_pad, zero, cdd_ref[...].astype(jnp.float32))
    c_ds = jnp.where(d_pad, zero, cds_ref[...].astype(jnp.float32))

    w1 = w1_ref[0, :].astype(jnp.bfloat16)
    b1 = b1_ref[0, :].astype(jnp.bfloat16)
    w2d = w2d_ref[...]
    b2 = b2_ref[0, :]
    b2c = jnp.concatenate([b2, b2])

    def encode(c_self, c_cross, out_ref):
        a1 = c_self.astype(jnp.bfloat16)[:, :, None] * w1 + b1
        a2 = c_cross.astype(jnp.bfloat16)[:, :, None] * w1 + b1
        h = jax.nn.relu(jnp.concatenate(
            [a1[:, :NH], a2[:, :NH], a1[:, NH:], a2[:, NH:]],
            axis=2))
        y = jnp.dot(h.reshape(BB * NH, 4 * FEAT), w2d,
                    preferred_element_type=jnp.float32)
        y = y + 2.0 * b2c[None, :]
        out_ref[...] = y.reshape(BB, NH, 2 * FEAT)[:, :N // 2, :]

    encode(c_ss, c_sd, out_src_ref)
    encode(c_dd, c_ds, out_dst_ref)


def _tc_encode(css, csd, cdd, cds, src_p, dst_p, w1, b1, w2d, b2):
    grid = B // BB
    cspec = pl.BlockSpec((BB, NP), lambda i: (i, 0))
    wspec = pl.BlockSpec((1, FEAT), lambda i: (0, 0))
    out_shape = [
        jax.ShapeDtypeStruct((B, N // 2, 2 * FEAT), jnp.float32),
        jax.ShapeDtypeStruct((B, N // 2, 2 * FEAT), jnp.float32),
    ]
    f = pl.pallas_call(
        _encode_tc_kernel,
        grid=(grid,),
        in_specs=[cspec, cspec, cspec, cspec, cspec, cspec,
                  wspec, wspec,
                  pl.BlockSpec((4 * FEAT, 2 * FEAT), lambda i: (0, 0)),
                  wspec],
        out_specs=[
            pl.BlockSpec((BB, N // 2, 2 * FEAT), lambda i: (i, 0, 0)),
            pl.BlockSpec((BB, N // 2, 2 * FEAT), lambda i: (i, 0, 0)),
        ],
        out_shape=out_shape,
    )
    o_s, o_d = f(css, csd, cdd, cds, src_p, dst_p, w1, b1, w2d, b2)
    return o_s.reshape(B, N, FEAT), o_d.reshape(B, N, FEAT)


@jax.jit
def _run(src, dst, w1, b1, w2, b2):
    pad_s = jnp.full((B, NP - N), SENT_S, jnp.int32)
    pad_d = jnp.full((B, NP - N), SENT_D, jnp.int32)
    perm = jnp.concatenate([jnp.arange(0, NP, 2), jnp.arange(1, NP, 2)])
    src_p = jnp.concatenate([src, pad_s], axis=1)[:, perm]
    dst_p = jnp.concatenate([dst, pad_d], axis=1)[:, perm]
    w2s = jnp.concatenate([w2, w2], axis=0).astype(jnp.bfloat16)
    zbk = jnp.zeros((2 * FEAT, FEAT), jnp.bfloat16)
    w2d = jnp.concatenate(
        [jnp.concatenate([w2s, zbk], axis=1),
         jnp.concatenate([zbk, w2s], axis=1)], axis=0)
    css, csd, cdd, cds = _sc_counts(src_p, dst_p)
    return _tc_encode(css, csd, cdd, cds, src_p, dst_p, w1, b1, w2d, b2)


def kernel(src_neighbour_nodes_ids, dst_neighbour_nodes_ids, W1, b1, W2, b2):
    w1 = W1.reshape(1, FEAT)
    b1r = b1.reshape(1, FEAT)
    b2r = b2.reshape(1, FEAT)
    out_s, out_d = _run(src_neighbour_nodes_ids, dst_neighbour_nodes_ids,
                        w1, b1r, W2, b2r)
    return (out_s, out_d)
